# baseline (speedup 1.0000x reference)
import functools

import jax
import jax.numpy as jnp
from jax import lax
from jax.experimental import pallas as pl
from jax.experimental.pallas import tpu as pltpu
from jax.experimental.pallas import tpu_sc as plsc

_N = 10000
_E = 320000
_D = 128
_DG = 128
_NC = 2
_NS = 16
_NW = _NC * _NS
_CHUNK = 128
_NBUF = 2
_KC = -(-_E // (_NW * _CHUNK * _NBUF)) * _NBUF
_EP = _KC * _NW * _CHUNK
_RPT = -(-(_N + 1) // (_NS * 8)) * 8
_NR = _RPT * _NS


def _sc_mesh():
    return plsc.VectorSubcoreMesh(
        core_axis_name="c", subcore_axis_name="s",
        num_cores=_NC, num_subcores=_NS)


def _sc_degree(srcw, zerosg, onesg):
    @functools.partial(
        pl.kernel,
        out_type=jax.ShapeDtypeStruct((_NC, _NR, _DG), jnp.float32),
        mesh=_sc_mesh(),
        scratch_types=[
            pltpu.VMEM((_KC, _CHUNK), jnp.int32),
            pltpu.VMEM((_CHUNK, _DG), jnp.float32),
            pltpu.VMEM_SHARED((_NR, _DG), jnp.float32),
        ],
    )
    def deg_kernel(srcw_hbm, zeros_hbm, ones_hbm, out_hbm, src_v, ones_v, acc_sh):
        c = lax.axis_index("c")
        s = lax.axis_index("s")
        wid = c * _NS + s
        pltpu.sync_copy(srcw_hbm.at[wid], src_v)
        pltpu.sync_copy(ones_hbm, ones_v)
        pltpu.sync_copy(zeros_hbm.at[pl.ds(s * _RPT, _RPT)],
                        acc_sh.at[pl.ds(s * _RPT, _RPT)])
        plsc.subcore_barrier()

        def body(k, carry):
            pltpu.sync_copy(ones_v, acc_sh.at[src_v.at[k]], add=True)
            return carry

        lax.fori_loop(0, _KC, body, 0)
        plsc.subcore_barrier()
        pltpu.sync_copy(acc_sh.at[pl.ds(s * _RPT, _RPT)],
                        out_hbm.at[c, pl.ds(s * _RPT, _RPT)])

    return deg_kernel(srcw, zerosg, onesg)


_BLK = 8
_NBLK = _KC // _BLK
_KSRC = _KC + _NBUF
_KDST = (_NBLK + 2) * _BLK


def _sc_scatter(y, srcw, dstw, zerosd):
    @functools.partial(
        pl.kernel,
        out_type=jax.ShapeDtypeStruct((_NC, _NR, _D), jnp.float32),
        mesh=_sc_mesh(),
        scratch_types=[
            pltpu.VMEM((_KSRC, _CHUNK), jnp.int32),
            pltpu.VMEM((2, _BLK, _CHUNK), jnp.int32),
            pltpu.VMEM((_NBUF, _CHUNK, _D), jnp.float32),
            pltpu.VMEM_SHARED((_NR, _D), jnp.float32),
        ] + [pltpu.SemaphoreType.DMA] * (_NBUF + 2),
    )
    def scat_kernel(y_hbm, srcw_hbm, dstw_hbm, zeros_hbm, out_hbm,
                    src_v, dst_v, rows_v, acc_sh, *sems):
        gsems = sems[:_NBUF]
        dsems = sems[_NBUF:]
        c = lax.axis_index("c")
        s = lax.axis_index("s")
        wid = c * _NS + s
        pltpu.sync_copy(srcw_hbm.at[wid], src_v)
        pltpu.sync_copy(zeros_hbm.at[pl.ds(s * _RPT, _RPT)],
                        acc_sh.at[pl.ds(s * _RPT, _RPT)])
        plsc.subcore_barrier()

        dbase = wid * _KDST

        for bb in range(2):
            pltpu.async_copy(dstw_hbm.at[pl.ds(dbase + bb * _BLK, _BLK)],
                             dst_v.at[bb], dsems[bb])
        for b in range(_NBUF):
            pltpu.async_copy(y_hbm.at[src_v.at[b]], rows_v.at[b], gsems[b])

        def body(sb, carry):
            for bb in range(2):
                blk = sb * 2 + bb
                pltpu.make_async_copy(
                    dstw_hbm.at[pl.ds(dbase, _BLK)], dst_v.at[bb],
                    dsems[bb]).wait()
                for j in range(_BLK):
                    b = j % _NBUF
                    k = blk * _BLK + j
                    pltpu.make_async_copy(
                        y_hbm.at[src_v.at[k]], rows_v.at[b], gsems[b]).wait()
                    pltpu.sync_copy(rows_v.at[b], acc_sh.at[dst_v.at[bb, j]],
                                    add=True)
                    pltpu.async_copy(y_hbm.at[src_v.at[k + _NBUF]],
                                     rows_v.at[b], gsems[b])
                pltpu.async_copy(
                    dstw_hbm.at[pl.ds(dbase + (blk + 2) * _BLK, _BLK)],
                    dst_v.at[bb], dsems[bb])
            return carry

        lax.fori_loop(0, _NBLK // 2, body, 0)

        for b in range(_NBUF):
            pltpu.make_async_copy(
                y_hbm.at[src_v.at[_KC + b]], rows_v.at[b], gsems[b]).wait()
        for bb in range(2):
            pltpu.make_async_copy(
                dstw_hbm.at[pl.ds(dbase, _BLK)], dst_v.at[bb],
                dsems[bb]).wait()

        plsc.subcore_barrier()
        pltpu.sync_copy(acc_sh.at[pl.ds(s * _RPT, _RPT)],
                        out_hbm.at[c, pl.ds(s * _RPT, _RPT)])

    return scat_kernel(y, srcw, dstw, zerosd)


def _dis_column(deg_ref):
    deg = deg_ref[0, :, 0:1] + deg_ref[1, :, 0:1]
    dis = jnp.where(deg > 0, lax.rsqrt(deg), 0.0)
    return lax.slice(dis, (0, 0), (_N, 1))


def _tc_first(x, w0, w1, b, degp):
    def body(x_ref, w0_ref, w1_ref, b_ref, deg_ref, p_ref, y_ref):
        dis_n = _dis_column(deg_ref)
        xv = x_ref[...]
        p_ref[...] = jnp.dot(xv, w0_ref[...],
                             preferred_element_type=jnp.float32) + b_ref[...]
        y_ref[0:_N, :] = jnp.dot(xv * dis_n, w1_ref[...],
                                 preferred_element_type=jnp.float32)
        y_ref[_N:_NR, :] = jnp.zeros((_NR - _N, _D), jnp.float32)

    return pl.pallas_call(
        body,
        out_shape=(jax.ShapeDtypeStruct((_N, _D), jnp.float32),
                   jax.ShapeDtypeStruct((_NR, _D), jnp.float32)),
    )(x, w0, w1, b, degp)


def _tc_mid(p1, sp, degp, w0, w1, b):
    def body(p1_ref, sp_ref, deg_ref, w0_ref, w1_ref, b_ref, p_ref, y_ref):
        dis_n = _dis_column(deg_ref)
        ssum = sp_ref[0, 0:_N, :] + sp_ref[1, 0:_N, :]
        h = jnp.maximum(p1_ref[...] - dis_n * ssum, 0.0)
        p_ref[...] = jnp.dot(h, w0_ref[...],
                             preferred_element_type=jnp.float32) + b_ref[...]
        y_ref[0:_N, :] = jnp.dot(h * dis_n, w1_ref[...],
                                 preferred_element_type=jnp.float32)
        y_ref[_N:_NR, :] = jnp.zeros((_NR - _N, _D), jnp.float32)

    return pl.pallas_call(
        body,
        out_shape=(jax.ShapeDtypeStruct((_N, _D), jnp.float32),
                   jax.ShapeDtypeStruct((_NR, _D), jnp.float32)),
    )(p1, sp, degp, w0, w1, b)


def _tc_last(p2, sp, degp):
    def body(p2_ref, sp_ref, deg_ref, out_ref):
        dis_n = _dis_column(deg_ref)
        ssum = sp_ref[0, 0:_N, :] + sp_ref[1, 0:_N, :]
        out_ref[...] = jnp.maximum(p2_ref[...] - dis_n * ssum, 0.0)

    return pl.pallas_call(
        body,
        out_shape=jax.ShapeDtypeStruct((_N, _D), jnp.float32),
    )(p2, sp, degp)


def kernel(x, edge_index, W1_0, W1_1, b1, W2_0, W2_1, b2):
    src = edge_index[0].astype(jnp.int32)
    dst = edge_index[1].astype(jnp.int32)
    fill = jnp.full((_EP - _E,), _N, jnp.int32)
    src80 = jnp.concatenate([src, fill]).reshape(_NW, _KC, _CHUNK)
    dst80 = jnp.concatenate([dst, fill]).reshape(_NW, _KC, _CHUNK)
    srcw = jnp.concatenate(
        [src80, jnp.full((_NW, _KSRC - _KC, _CHUNK), _N, jnp.int32)], axis=1)
    dstw = jnp.concatenate(
        [dst80, jnp.full((_NW, _KDST - _KC, _CHUNK), _N, jnp.int32)],
        axis=1).reshape(_NW * _KDST, _CHUNK)
    srcw_deg = src80
    zerosd = jnp.zeros((_NR, _D), jnp.float32)
    zerosg = jnp.zeros((_NR, _DG), jnp.float32)
    onesg = jnp.ones((_CHUNK, _DG), jnp.float32)

    degp = _sc_degree(srcw_deg, zerosg, onesg)
    p1, y1 = _tc_first(x, W1_0, W1_1, b1.reshape(1, _D), degp)
    sp1 = _sc_scatter(y1, srcw, dstw, zerosd)
    p2, y2 = _tc_mid(p1, sp1, degp, W2_0, W2_1, b2.reshape(1, _D))
    sp2 = _sc_scatter(y2, srcw, dstw, zerosd)
    return _tc_last(p2, sp2, degp)


# speedup vs baseline: 2.0492x; 2.0492x over previous
import functools

import jax
import jax.numpy as jnp
from jax import lax
from jax.experimental import pallas as pl
from jax.experimental.pallas import tpu as pltpu
from jax.experimental.pallas import tpu_sc as plsc

_N = 10000
_E = 320000
_D = 128
_NC = 2
_NS = 16
_NW = _NC * _NS
_CHUNK = 128
_KC = -(-_E // (_NW * _CHUNK))
_EP = _KC * _NW * _CHUNK
_RPT = -(-(_N + 1) // (_NS * 8)) * 8
_NR = _RPT * _NS


def _sc_mesh():
    return plsc.VectorSubcoreMesh(
        core_axis_name="c", subcore_axis_name="s",
        num_cores=_NC, num_subcores=_NS)


def _sc_degree(srcw, zerosd, onesd):
    @functools.partial(
        pl.kernel,
        out_type=jax.ShapeDtypeStruct((_NC, _NR, _D), jnp.float32),
        mesh=_sc_mesh(),
        scratch_types=[
            pltpu.VMEM((_KC, _CHUNK), jnp.int32),
            pltpu.VMEM((_CHUNK, _D), jnp.float32),
            pltpu.VMEM_SHARED((_NR, _D), jnp.float32),
        ],
    )
    def deg_kernel(srcw_hbm, zeros_hbm, ones_hbm, out_hbm, src_v, ones_v, acc_sh):
        c = lax.axis_index("c")
        s = lax.axis_index("s")
        wid = c * _NS + s
        pltpu.sync_copy(srcw_hbm.at[wid], src_v)
        pltpu.sync_copy(ones_hbm, ones_v)
        pltpu.sync_copy(zeros_hbm.at[pl.ds(s * _RPT, _RPT)],
                        acc_sh.at[pl.ds(s * _RPT, _RPT)])
        plsc.subcore_barrier()

        def body(k, carry):
            pltpu.sync_copy(ones_v, acc_sh.at[src_v.at[k]], add=True)
            return carry

        lax.fori_loop(0, _KC, body, 0)
        plsc.subcore_barrier()
        pltpu.sync_copy(acc_sh.at[pl.ds(s * _RPT, _RPT)],
                        out_hbm.at[c, pl.ds(s * _RPT, _RPT)])

    return deg_kernel(srcw, zerosd, onesd)


def _sc_scatter(y, srcw, dstw, zerosd):
    @functools.partial(
        pl.kernel,
        out_type=jax.ShapeDtypeStruct((_NC, _NR, _D), jnp.float32),
        mesh=_sc_mesh(),
        scratch_types=[
            pltpu.VMEM((_KC, _CHUNK), jnp.int32),
            pltpu.VMEM((_KC, _CHUNK), jnp.int32),
            pltpu.VMEM((_CHUNK, _D), jnp.float32),
            pltpu.VMEM_SHARED((_NR, _D), jnp.float32),
            pltpu.SemaphoreType.DMA,
        ],
    )
    def scat_kernel(y_hbm, srcw_hbm, dstw_hbm, zeros_hbm, out_hbm,
                    src_v, dst_v, rows_v, acc_sh, sem):
        c = lax.axis_index("c")
        s = lax.axis_index("s")
        wid = c * _NS + s
        pltpu.sync_copy(srcw_hbm.at[wid], src_v)
        pltpu.sync_copy(dstw_hbm.at[wid], dst_v)
        pltpu.sync_copy(zeros_hbm.at[pl.ds(s * _RPT, _RPT)],
                        acc_sh.at[pl.ds(s * _RPT, _RPT)])
        plsc.subcore_barrier()

        def body(k, carry):
            pltpu.async_copy(y_hbm.at[src_v.at[k]], rows_v, sem).wait()
            pltpu.sync_copy(rows_v, acc_sh.at[dst_v.at[k]], add=True)
            return carry

        lax.fori_loop(0, _KC, body, 0)
        plsc.subcore_barrier()
        pltpu.sync_copy(acc_sh.at[pl.ds(s * _RPT, _RPT)],
                        out_hbm.at[c, pl.ds(s * _RPT, _RPT)])

    return scat_kernel(y, srcw, dstw, zerosd)


def _dis_column(deg_ref):
    deg = deg_ref[0, :, 0:1] + deg_ref[1, :, 0:1]
    dis = jnp.where(deg > 0, lax.rsqrt(deg), 0.0)
    return lax.slice(dis, (0, 0), (_N, 1))


def _tc_first(x, w0, w1, b, degp):
    def body(x_ref, w0_ref, w1_ref, b_ref, deg_ref, p_ref, y_ref):
        dis_n = _dis_column(deg_ref)
        xv = x_ref[...]
        p_ref[...] = jnp.dot(xv, w0_ref[...],
                             preferred_element_type=jnp.float32) + b_ref[...]
        y_ref[0:_N, :] = jnp.dot(xv * dis_n, w1_ref[...],
                                 preferred_element_type=jnp.float32)
        y_ref[_N:_NR, :] = jnp.zeros((_NR - _N, _D), jnp.float32)

    return pl.pallas_call(
        body,
        out_shape=(jax.ShapeDtypeStruct((_N, _D), jnp.float32),
                   jax.ShapeDtypeStruct((_NR, _D), jnp.float32)),
    )(x, w0, w1, b, degp)


def _tc_mid(p1, sp, degp, w0, w1, b):
    def body(p1_ref, sp_ref, deg_ref, w0_ref, w1_ref, b_ref, p_ref, y_ref):
        dis_n = _dis_column(deg_ref)
        ssum = sp_ref[0, 0:_N, :] + sp_ref[1, 0:_N, :]
        h = jnp.maximum(p1_ref[...] - dis_n * ssum, 0.0)
        p_ref[...] = jnp.dot(h, w0_ref[...],
                             preferred_element_type=jnp.float32) + b_ref[...]
        y_ref[0:_N, :] = jnp.dot(h * dis_n, w1_ref[...],
                                 preferred_element_type=jnp.float32)
        y_ref[_N:_NR, :] = jnp.zeros((_NR - _N, _D), jnp.float32)

    return pl.pallas_call(
        body,
        out_shape=(jax.ShapeDtypeStruct((_N, _D), jnp.float32),
                   jax.ShapeDtypeStruct((_NR, _D), jnp.float32)),
    )(p1, sp, degp, w0, w1, b)


def _tc_last(p2, sp, degp):
    def body(p2_ref, sp_ref, deg_ref, out_ref):
        dis_n = _dis_column(deg_ref)
        ssum = sp_ref[0, 0:_N, :] + sp_ref[1, 0:_N, :]
        out_ref[...] = jnp.maximum(p2_ref[...] - dis_n * ssum, 0.0)

    return pl.pallas_call(
        body,
        out_shape=jax.ShapeDtypeStruct((_N, _D), jnp.float32),
    )(p2, sp, degp)


def kernel(x, edge_index, W1_0, W1_1, b1, W2_0, W2_1, b2):
    src = edge_index[0].astype(jnp.int32)
    dst = edge_index[1].astype(jnp.int32)
    fill = jnp.full((_EP - _E,), _N, jnp.int32)
    srcw = jnp.concatenate([src, fill]).reshape(_NW, _KC, _CHUNK)
    dstw = jnp.concatenate([dst, fill]).reshape(_NW, _KC, _CHUNK)
    zerosd = jnp.zeros((_NR, _D), jnp.float32)
    onesd = jnp.ones((_CHUNK, _D), jnp.float32)

    degp = _sc_degree(srcw, zerosd, onesd)
    p1, y1 = _tc_first(x, W1_0, W1_1, b1.reshape(1, _D), degp)
    sp1 = _sc_scatter(y1, srcw, dstw, zerosd)
    p2, y2 = _tc_mid(p1, sp1, degp, W2_0, W2_1, b2.reshape(1, _D))
    sp2 = _sc_scatter(y2, srcw, dstw, zerosd)
    return _tc_last(p2, sp2, degp)
